# Initial kernel scaffold; baseline (speedup 1.0000x reference)
import jax
import jax.numpy as jnp
from jax.experimental import pallas as pl


def kernel(keypoint_embeddings, Wi, bi, gi, bbi, ecW0, ecg0, ecb0, bng0, bnb0, adW0, adb0, adg0, adbb0, aeW0, aeg0, aeb0, aaW0, aab0, ecW1, ecg1, ecb1, bng1, bnb1, adW1, adb1, adg1, adbb1, aeW1, aeg1, aeb1, aaW1, aab1):
    raise NotImplementedError("write your pallas kernel here")


# speedup vs baseline: 4.5465x; 4.5465x over previous
import functools

import jax
import jax.numpy as jnp
import numpy as np
from jax.experimental import pallas as pl

_CONN = [[15, 13], [13, 11], [16, 14], [14, 12], [11, 12], [5, 11], [6, 12],
         [5, 6], [5, 7], [6, 8], [7, 9], [8, 10], [1, 2], [0, 1], [0, 2],
         [1, 3], [2, 4], [3, 5], [4, 6]]
_K = 17
_HID = 256
_INTER = 64
_L = 2
_B = 256
_EPS = 1e-5

_ROW, _COL = [], []
for _s, _d in _CONN:
    _ROW += [_s, _d]
    _COL += [_d, _s]
_E = len(_ROW)
_G = len(_CONN)
_INC = [[e for e, r in enumerate(_ROW) if r == n] for n in range(_K)]

_PREC = jax.lax.Precision.HIGHEST


def _silu(x):
    return x * jax.nn.sigmoid(x)


def _bn_ax0(x3, g, b):
    m = x3.mean(0)
    v = (x3 * x3).mean(0) - m * m
    return (x3 - m) * jax.lax.rsqrt(v + _EPS) * g + b


def _dot(a, w):
    return jnp.dot(a, w, preferred_element_type=jnp.float32, precision=_PREC)


def _gnn_kernel(x_ref, wiT_ref, bi_ref, gi_ref, bbi_ref, *lrefs, out_ref):
    n, s, c = x_ref.shape
    x3 = x_ref[...]
    h = _dot(x3.reshape(n * s, c), wiT_ref[...]) + bi_ref[...]
    x3 = _silu(_bn_ax0(h.reshape(n, s, c), gi_ref[...], bbi_ref[...]))

    for i in range(_L):
        (w1dT, w2T, ecg, ecb, bng, bnb, adWT, adb, adg, adbb,
         v1dT, v2T, aeg, aeb, aaWT, aab) = lrefs[16 * i:16 * (i + 1)]
        xr = x3
        xf = x3.reshape(n * s, c)
        a3 = _dot(xf, w1dT[...]).reshape(n, s, c)
        b3 = _dot(xf, w2T[...]).reshape(n, s, c)
        ef3 = jnp.stack([a3[r] + b3[q] for r, q in zip(_ROW, _COL)], 0)
        me = ef3.mean(0)
        ve = (ef3 * ef3).mean(0) - me * me
        h3 = _silu((ef3 - me) * jax.lax.rsqrt(ve + _EPS) * ecg[...] + ecb[...])
        out3 = jnp.stack([sum(h3[e] for e in _INC[nn]) for nn in range(n)], 0)
        x3 = _bn_ax0(out3, bng[...], bnb[...])

        xdf = _dot(x3.reshape(n * s, c), adWT[...]) + adb[...]
        xd3 = _silu(_bn_ax0(xdf.reshape(n, s, _INTER), adg[...], adbb[...]))
        xs3 = jnp.stack([(xd3[a] + xd3[b]) * 0.5 for a, b in _CONN], 0)
        xsf = xs3.reshape(_G * s, _INTER)
        a2 = _dot(xsf, v1dT[...]).reshape(_G, s, _INTER)
        b2 = _dot(xsf, v2T[...]).reshape(_G, s, _INTER)
        m2a = a2.mean(0)
        m2b = b2.mean(0)
        v2 = ((a2 * a2).mean(0) - m2a * m2a) + ((b2 * b2).mean(0) - m2b * m2b)
        scale = aeg[...] * jax.lax.rsqrt(v2 + _EPS)
        shift = aeb[...] - (m2a + m2b) * scale
        attg = _silu(a2[:, None] * scale + (b2[None, :] * scale + shift)).sum(1)
        att = jax.nn.sigmoid(_dot(attg.reshape(_G * s, _INTER), aaWT[...]) + aab[...])
        attm = att.reshape(_G, s, c).mean(0)
        x3 = _silu(x3 * attm + xr)
    out_ref[...] = x3


@functools.partial(jax.jit, static_argnames=())
def kernel(keypoint_embeddings, Wi, bi, gi, bbi,
           ecW0, ecg0, ecb0, bng0, bnb0, adW0, adb0, adg0, adbb0,
           aeW0, aeg0, aeb0, aaW0, aab0,
           ecW1, ecg1, ecb1, bng1, bnb1, adW1, adb1, adg1, adbb1,
           aeW1, aeg1, aeb1, aaW1, aab1):
    S = 64
    x = jnp.transpose(keypoint_embeddings, (1, 0, 2))
    b = x.shape[1]

    def row(v):
        return v.reshape(1, -1)

    ops = [x, Wi.T, row(bi), row(gi), row(bbi)]
    for (ecW, ecg, ecb, bng, bnb, adW, adb, adg, adbb,
         aeW, aeg, aeb, aaW, aab) in (
            (ecW0, ecg0, ecb0, bng0, bnb0, adW0, adb0, adg0, adbb0,
             aeW0, aeg0, aeb0, aaW0, aab0),
            (ecW1, ecg1, ecb1, bng1, bnb1, adW1, adb1, adg1, adbb1,
             aeW1, aeg1, aeb1, aaW1, aab1)):
        ops += [(ecW[:, :_HID] - ecW[:, _HID:]).T, ecW[:, _HID:].T,
                row(ecg), row(ecb), row(bng), row(bnb),
                adW.T, row(adb), row(adg), row(adbb),
                (aeW[:, :_INTER] - aeW[:, _INTER:]).T, aeW[:, _INTER:].T,
                row(aeg), row(aeb), aaW.T, row(aab)]

    full = lambda arr: pl.BlockSpec(arr.shape, lambda i: (0,) * arr.ndim)
    in_specs = [pl.BlockSpec((_K, S, _HID), lambda i: (0, i, 0))]
    in_specs += [full(o) for o in ops[1:]]

    out = pl.pallas_call(
        lambda *refs: _gnn_kernel(*refs[:-1], out_ref=refs[-1]),
        grid=(b // S,),
        in_specs=in_specs,
        out_specs=pl.BlockSpec((_K, S, _HID), lambda i: (0, i, 0)),
        out_shape=jax.ShapeDtypeStruct((_K, b, _HID), jnp.float32),
    )(*ops)
    return jnp.transpose(out, (1, 0, 2))


# speedup vs baseline: 7.9318x; 1.7446x over previous
import functools

import jax
import jax.numpy as jnp
import numpy as np
from jax.experimental import pallas as pl

_CONN = [[15, 13], [13, 11], [16, 14], [14, 12], [11, 12], [5, 11], [6, 12],
         [5, 6], [5, 7], [6, 8], [7, 9], [8, 10], [1, 2], [0, 1], [0, 2],
         [1, 3], [2, 4], [3, 5], [4, 6]]
_K = 17
_HID = 256
_INTER = 64
_L = 2
_B = 256
_EPS = 1e-5

_ROW, _COL = [], []
for _s, _d in _CONN:
    _ROW += [_s, _d]
    _COL += [_d, _s]
_E = len(_ROW)
_G = len(_CONN)
_INC = [[e for e, r in enumerate(_ROW) if r == n] for n in range(_K)]

_PREC = jax.lax.Precision.DEFAULT


def _silu(x):
    return x * jax.nn.sigmoid(x)


def _bn_ax0(x3, g, b):
    m = x3.mean(0)
    v = (x3 * x3).mean(0) - m * m
    return (x3 - m) * jax.lax.rsqrt(v + _EPS) * g + b


def _dot(a, w):
    return jnp.dot(a, w, preferred_element_type=jnp.float32, precision=_PREC)


def _gnn_kernel(x_ref, wiT_ref, bi_ref, gi_ref, bbi_ref, *lrefs, out_ref):
    n, s, c = x_ref.shape
    x3 = x_ref[...]
    h = _dot(x3.reshape(n * s, c), wiT_ref[...]) + bi_ref[...]
    x3 = _silu(_bn_ax0(h.reshape(n, s, c), gi_ref[...], bbi_ref[...]))

    for i in range(_L):
        (w1dT, w2T, ecg, ecb, bng, bnb, adWT, adb, adg, adbb,
         v1dT, v2T, aeg, aeb, aaWT, aab) = lrefs[16 * i:16 * (i + 1)]
        xr = x3
        xf = x3.reshape(n * s, c)
        a3 = _dot(xf, w1dT[...]).reshape(n, s, c)
        b3 = _dot(xf, w2T[...]).reshape(n, s, c)
        ef3 = jnp.stack([a3[r] + b3[q] for r, q in zip(_ROW, _COL)], 0)
        me = ef3.mean(0)
        ve = (ef3 * ef3).mean(0) - me * me
        h3 = _silu((ef3 - me) * jax.lax.rsqrt(ve + _EPS) * ecg[...] + ecb[...])
        out3 = jnp.stack([sum(h3[e] for e in _INC[nn]) for nn in range(n)], 0)
        x3 = _bn_ax0(out3, bng[...], bnb[...])

        xdf = _dot(x3.reshape(n * s, c), adWT[...]) + adb[...]
        xd3 = _silu(_bn_ax0(xdf.reshape(n, s, _INTER), adg[...], adbb[...]))
        xs3 = jnp.stack([(xd3[a] + xd3[b]) * 0.5 for a, b in _CONN], 0)
        xsf = xs3.reshape(_G * s, _INTER)
        a2 = _dot(xsf, v1dT[...]).reshape(_G, s, _INTER)
        b2 = _dot(xsf, v2T[...]).reshape(_G, s, _INTER)
        m2a = a2.mean(0)
        m2b = b2.mean(0)
        v2 = ((a2 * a2).mean(0) - m2a * m2a) + ((b2 * b2).mean(0) - m2b * m2b)
        scale = aeg[...] * jax.lax.rsqrt(v2 + _EPS)
        shift = aeb[...] - (m2a + m2b) * scale
        attg = _silu(a2[:, None] * scale + (b2[None, :] * scale + shift)).sum(1)
        att = jax.nn.sigmoid(_dot(attg.reshape(_G * s, _INTER), aaWT[...]) + aab[...])
        attm = att.reshape(_G, s, c).mean(0)
        x3 = _silu(x3 * attm + xr)
    out_ref[...] = x3


@functools.partial(jax.jit, static_argnames=())
def kernel(keypoint_embeddings, Wi, bi, gi, bbi,
           ecW0, ecg0, ecb0, bng0, bnb0, adW0, adb0, adg0, adbb0,
           aeW0, aeg0, aeb0, aaW0, aab0,
           ecW1, ecg1, ecb1, bng1, bnb1, adW1, adb1, adg1, adbb1,
           aeW1, aeg1, aeb1, aaW1, aab1):
    S = 64
    x = jnp.transpose(keypoint_embeddings, (1, 0, 2))
    b = x.shape[1]

    def row(v):
        return v.reshape(1, -1)

    ops = [x, Wi.T, row(bi), row(gi), row(bbi)]
    for (ecW, ecg, ecb, bng, bnb, adW, adb, adg, adbb,
         aeW, aeg, aeb, aaW, aab) in (
            (ecW0, ecg0, ecb0, bng0, bnb0, adW0, adb0, adg0, adbb0,
             aeW0, aeg0, aeb0, aaW0, aab0),
            (ecW1, ecg1, ecb1, bng1, bnb1, adW1, adb1, adg1, adbb1,
             aeW1, aeg1, aeb1, aaW1, aab1)):
        ops += [(ecW[:, :_HID] - ecW[:, _HID:]).T, ecW[:, _HID:].T,
                row(ecg), row(ecb), row(bng), row(bnb),
                adW.T, row(adb), row(adg), row(adbb),
                (aeW[:, :_INTER] - aeW[:, _INTER:]).T, aeW[:, _INTER:].T,
                row(aeg), row(aeb), aaW.T, row(aab)]

    full = lambda arr: pl.BlockSpec(arr.shape, lambda i: (0,) * arr.ndim)
    in_specs = [pl.BlockSpec((_K, S, _HID), lambda i: (0, i, 0))]
    in_specs += [full(o) for o in ops[1:]]

    out = pl.pallas_call(
        lambda *refs: _gnn_kernel(*refs[:-1], out_ref=refs[-1]),
        grid=(b // S,),
        in_specs=in_specs,
        out_specs=pl.BlockSpec((_K, S, _HID), lambda i: (0, i, 0)),
        out_shape=jax.ShapeDtypeStruct((_K, b, _HID), jnp.float32),
    )(*ops)
    return jnp.transpose(out, (1, 0, 2))


# speedup vs baseline: 8.1830x; 1.0317x over previous
import functools

import jax
import jax.numpy as jnp
import numpy as np
from jax.experimental import pallas as pl

_CONN = [[15, 13], [13, 11], [16, 14], [14, 12], [11, 12], [5, 11], [6, 12],
         [5, 6], [5, 7], [6, 8], [7, 9], [8, 10], [1, 2], [0, 1], [0, 2],
         [1, 3], [2, 4], [3, 5], [4, 6]]
_K = 17
_HID = 256
_INTER = 64
_L = 2
_B = 256
_EPS = 1e-5

_ROW, _COL = [], []
for _s, _d in _CONN:
    _ROW += [_s, _d]
    _COL += [_d, _s]
_E = len(_ROW)
_G = len(_CONN)
_INC = [[e for e, r in enumerate(_ROW) if r == n] for n in range(_K)]

_PREC = jax.lax.Precision.DEFAULT


def _silu(x):
    return x * jax.nn.sigmoid(x)


def _bn_ax0(x3, g, b):
    m = x3.mean(0)
    v = (x3 * x3).mean(0) - m * m
    return (x3 - m) * jax.lax.rsqrt(v + _EPS) * g + b


def _dot(a, w):
    return jnp.dot(a, w, preferred_element_type=jnp.float32, precision=_PREC)


def _gnn_kernel(x_ref, wiT_ref, bi_ref, gi_ref, bbi_ref, *lrefs, out_ref):
    n, s, c = x_ref.shape
    x3 = x_ref[...]
    h = _dot(x3.reshape(n * s, c), wiT_ref[...]) + bi_ref[...]
    x3 = _silu(_bn_ax0(h.reshape(n, s, c), gi_ref[...], bbi_ref[...]))

    for i in range(_L):
        (w1dT, w2T, ecg, ecb, bng, bnb, adWT, adb, adg, adbb,
         v1dT, v2T, aeg, aeb, aaWT, aab) = lrefs[16 * i:16 * (i + 1)]
        xr = x3
        xf = x3.reshape(n * s, c)
        a3 = _dot(xf, w1dT[...]).reshape(n, s, c)
        b3 = _dot(xf, w2T[...]).reshape(n, s, c)
        ef3 = jnp.stack([a3[r] + b3[q] for r, q in zip(_ROW, _COL)], 0)
        me = ef3.mean(0)
        ve = (ef3 * ef3).mean(0) - me * me
        h3 = _silu((ef3 - me) * jax.lax.rsqrt(ve + _EPS) * ecg[...] + ecb[...])
        out3 = jnp.stack([sum(h3[e] for e in _INC[nn]) for nn in range(n)], 0)
        x3 = _bn_ax0(out3, bng[...], bnb[...])

        xdf = _dot(x3.reshape(n * s, c), adWT[...]) + adb[...]
        xd3 = _silu(_bn_ax0(xdf.reshape(n, s, _INTER), adg[...], adbb[...]))
        xs3 = jnp.stack([(xd3[a] + xd3[b]) * 0.5 for a, b in _CONN], 0)
        xsf = xs3.reshape(_G * s, _INTER)
        a2 = _dot(xsf, v1dT[...]).reshape(_G, s, _INTER)
        b2 = _dot(xsf, v2T[...]).reshape(_G, s, _INTER)
        m2a = a2.mean(0)
        m2b = b2.mean(0)
        v2 = ((a2 * a2).mean(0) - m2a * m2a) + ((b2 * b2).mean(0) - m2b * m2b)
        scale = aeg[...] * jax.lax.rsqrt(v2 + _EPS)
        shift = aeb[...] - (m2a + m2b) * scale
        attg = _silu(a2[:, None] * scale + (b2[None, :] * scale + shift)).sum(1)
        att = jax.nn.sigmoid(_dot(attg.reshape(_G * s, _INTER), aaWT[...]) + aab[...])
        attm = att.reshape(_G, s, c).mean(0)
        x3 = _silu(x3 * attm + xr)
    out_ref[...] = x3


@functools.partial(jax.jit, static_argnames=())
def kernel(keypoint_embeddings, Wi, bi, gi, bbi,
           ecW0, ecg0, ecb0, bng0, bnb0, adW0, adb0, adg0, adbb0,
           aeW0, aeg0, aeb0, aaW0, aab0,
           ecW1, ecg1, ecb1, bng1, bnb1, adW1, adb1, adg1, adbb1,
           aeW1, aeg1, aeb1, aaW1, aab1):
    S = 128
    x = jnp.transpose(keypoint_embeddings, (1, 0, 2))
    b = x.shape[1]

    def row(v):
        return v.reshape(1, -1)

    ops = [x, Wi.T, row(bi), row(gi), row(bbi)]
    for (ecW, ecg, ecb, bng, bnb, adW, adb, adg, adbb,
         aeW, aeg, aeb, aaW, aab) in (
            (ecW0, ecg0, ecb0, bng0, bnb0, adW0, adb0, adg0, adbb0,
             aeW0, aeg0, aeb0, aaW0, aab0),
            (ecW1, ecg1, ecb1, bng1, bnb1, adW1, adb1, adg1, adbb1,
             aeW1, aeg1, aeb1, aaW1, aab1)):
        ops += [(ecW[:, :_HID] - ecW[:, _HID:]).T, ecW[:, _HID:].T,
                row(ecg), row(ecb), row(bng), row(bnb),
                adW.T, row(adb), row(adg), row(adbb),
                (aeW[:, :_INTER] - aeW[:, _INTER:]).T, aeW[:, _INTER:].T,
                row(aeg), row(aeb), aaW.T, row(aab)]

    full = lambda arr: pl.BlockSpec(arr.shape, lambda i: (0,) * arr.ndim)
    in_specs = [pl.BlockSpec((_K, S, _HID), lambda i: (0, i, 0))]
    in_specs += [full(o) for o in ops[1:]]

    out = pl.pallas_call(
        lambda *refs: _gnn_kernel(*refs[:-1], out_ref=refs[-1]),
        grid=(b // S,),
        in_specs=in_specs,
        out_specs=pl.BlockSpec((_K, S, _HID), lambda i: (0, i, 0)),
        out_shape=jax.ShapeDtypeStruct((_K, b, _HID), jnp.float32),
    )(*ops)
    return jnp.transpose(out, (1, 0, 2))
